# Initial kernel scaffold; baseline (speedup 1.0000x reference)
import jax
import jax.numpy as jnp
from jax.experimental import pallas as pl


def kernel(normalized_x, edge_index, W1, b1, W3, b3):
    raise NotImplementedError("write your pallas kernel here")


# speedup vs baseline: 330.2813x; 330.2813x over previous
import functools

import jax
import jax.numpy as jnp
from jax import lax
from jax.experimental import pallas as pl
from jax.experimental.pallas import tpu as pltpu
from jax.experimental.pallas import tpu_sc as plsc

N_NODES = 100000
N_EDGES = 3200000
NC = 2
NS = 16
NW = NC * NS
LANE = 16
N_PAD = 100352
SLICE = N_PAD // NS
VECS = SLICE // LANE
ROW = 128
CH = 100
E_ROWS = N_EDGES // ROW
N_CHUNKS = E_ROWS // CH

_mesh = lambda: plsc.VectorSubcoreMesh(core_axis_name="c", subcore_axis_name="s")


def _rsqrt(d):
    i = lax.bitcast_convert_type(d, jnp.int32)
    i = jnp.int32(0x5F3759DF) - (i >> 1)
    y = lax.bitcast_convert_type(i, jnp.float32)
    for _ in range(3):
        y = y * (1.5 - 0.5 * d * y * y)
    return y


def _n_my_chunks(wid):
    return (N_CHUNKS - 1 - wid) // NW + 1


@functools.partial(
    pl.kernel,
    mesh=_mesh(),
    out_type=jax.ShapeDtypeStruct((NC * N_PAD,), jnp.float32),
    scratch_types=[
        pltpu.VMEM((CH * ROW,), jnp.int32),
        pltpu.VMEM((CH * ROW,), jnp.float32),
        pltpu.VMEM((SLICE,), jnp.float32),
        pltpu.VMEM_SHARED((N_PAD,), jnp.float32),
    ],
)
def _deg_kernel(dst_hbm, ones_hbm, zeros_hbm, out_hbm, dst_v, ones_v, zbuf, acc):
    core = lax.axis_index("c")
    tid = lax.axis_index("s")
    wid = tid * NC + core
    off = tid * SLICE
    pltpu.sync_copy(zeros_hbm, zbuf)
    pltpu.sync_copy(zbuf, acc.at[pl.ds(off, SLICE)])
    pltpu.sync_copy(ones_hbm, ones_v)
    plsc.subcore_barrier()

    def ebody(i, carry):
        c = wid + i * NW
        pltpu.sync_copy(dst_hbm.at[pl.ds(c * CH * ROW, CH * ROW)], dst_v)
        pltpu.sync_copy(ones_v, acc.at[dst_v], add=True)
        return carry

    lax.fori_loop(0, _n_my_chunks(wid), ebody, 0)
    plsc.subcore_barrier()
    pltpu.sync_copy(acc.at[pl.ds(off, SLICE)], zbuf)
    pltpu.sync_copy(zbuf, out_hbm.at[pl.ds(core * N_PAD + off, SLICE)])


@functools.partial(
    pl.kernel,
    mesh=_mesh(),
    out_type=(
        jax.ShapeDtypeStruct((NC * N_PAD,), jnp.float32),
        jax.ShapeDtypeStruct((N_PAD,), jnp.float32),
    ),
    scratch_types=[
        pltpu.VMEM((CH * ROW,), jnp.int32),
        pltpu.VMEM((CH * ROW,), jnp.int32),
        pltpu.VMEM((CH * ROW,), jnp.float32),
        pltpu.VMEM((SLICE,), jnp.float32),
        pltpu.VMEM((SLICE,), jnp.float32),
        pltpu.VMEM((SLICE,), jnp.float32),
        pltpu.VMEM((SLICE,), jnp.float32),
        pltpu.VMEM((SLICE,), jnp.float32),
        pltpu.VMEM_SHARED((N_PAD,), jnp.float32),
        pltpu.VMEM_SHARED((N_PAD,), jnp.float32),
    ],
)
def _conv1_kernel(x_hbm, degp_hbm, src_hbm, dst_hbm, zeros_hbm,
                  g_hbm, dis_hbm,
                  src_v, dst_v, msg_v, a_v, b_v, x_v, dis_v, u_v, usp, acc):
    core = lax.axis_index("c")
    tid = lax.axis_index("s")
    wid = tid * NC + core
    off = tid * SLICE
    pltpu.sync_copy(zeros_hbm, u_v)
    pltpu.sync_copy(u_v, acc.at[pl.ds(off, SLICE)])
    pltpu.sync_copy(degp_hbm.at[pl.ds(off, SLICE)], a_v)
    pltpu.sync_copy(degp_hbm.at[pl.ds(N_PAD + off, SLICE)], b_v)
    pltpu.sync_copy(x_hbm.at[pl.ds(off, SLICE)], x_v)

    def nbody(k, carry):
        sl = pl.ds(k * LANE, LANE)
        d = a_v[sl] + b_v[sl] + 1.0
        y = _rsqrt(d)
        dis_v[sl] = y
        u_v[sl] = y * x_v[sl]
        return carry

    lax.fori_loop(0, VECS, nbody, 0)
    pltpu.sync_copy(u_v, usp.at[pl.ds(off, SLICE)])

    @pl.when(core == 0)
    def _():
        pltpu.sync_copy(dis_v, dis_hbm.at[pl.ds(off, SLICE)])

    plsc.subcore_barrier()

    def ebody(i, carry):
        c = wid + i * NW
        pltpu.sync_copy(src_hbm.at[pl.ds(c * CH * ROW, CH * ROW)], src_v)
        pltpu.sync_copy(dst_hbm.at[pl.ds(c * CH * ROW, CH * ROW)], dst_v)
        pltpu.sync_copy(usp.at[src_v], msg_v)
        pltpu.sync_copy(msg_v, acc.at[dst_v], add=True)
        return carry

    lax.fori_loop(0, _n_my_chunks(wid), ebody, 0)
    plsc.subcore_barrier()
    pltpu.sync_copy(acc.at[pl.ds(off, SLICE)], u_v)
    pltpu.sync_copy(u_v, g_hbm.at[pl.ds(core * N_PAD + off, SLICE)])


@functools.partial(
    pl.kernel,
    mesh=_mesh(),
    out_type=(
        jax.ShapeDtypeStruct((NC * N_PAD,), jnp.float32),
        jax.ShapeDtypeStruct((N_PAD,), jnp.float32),
    ),
    scratch_types=[
        pltpu.VMEM((CH * ROW,), jnp.int32),
        pltpu.VMEM((CH * ROW,), jnp.int32),
        pltpu.VMEM((CH * ROW,), jnp.float32),
        pltpu.VMEM((SLICE,), jnp.float32),
        pltpu.VMEM((SLICE,), jnp.float32),
        pltpu.VMEM((SLICE,), jnp.float32),
        pltpu.VMEM((SLICE,), jnp.float32),
        pltpu.VMEM((SLICE,), jnp.float32),
        pltpu.VMEM((LANE, LANE), jnp.float32),
        pltpu.VMEM((LANE, LANE), jnp.float32),
        pltpu.VMEM((LANE, LANE), jnp.float32),
        pltpu.VMEM_SHARED((N_PAD,), jnp.float32),
        pltpu.VMEM_SHARED((N_PAD,), jnp.float32),
    ],
)
def _conv2_kernel(x_hbm, dis_hbm, gp_hbm, src_hbm, dst_hbm, zeros_hbm,
                  w1_hbm, b1_hbm, w3_hbm,
                  g_hbm, w_hbm,
                  src_v, dst_v, msg_v, ga_v, gb_v, x_v, dis_v, w_v,
                  w1_v, b1_v, w3_v, wsp, acc):
    core = lax.axis_index("c")
    tid = lax.axis_index("s")
    wid = tid * NC + core
    off = tid * SLICE
    pltpu.sync_copy(zeros_hbm, w_v)
    pltpu.sync_copy(w_v, acc.at[pl.ds(off, SLICE)])
    pltpu.sync_copy(gp_hbm.at[pl.ds(off, SLICE)], ga_v)
    pltpu.sync_copy(gp_hbm.at[pl.ds(N_PAD + off, SLICE)], gb_v)
    pltpu.sync_copy(x_hbm.at[pl.ds(off, SLICE)], x_v)
    pltpu.sync_copy(dis_hbm.at[pl.ds(off, SLICE)], dis_v)
    pltpu.sync_copy(w1_hbm, w1_v)
    pltpu.sync_copy(b1_hbm, b1_v)
    pltpu.sync_copy(w3_hbm, w3_v)

    def nbody(k, carry):
        sl = pl.ds(k * LANE, LANE)
        y = dis_v[sl]
        s1 = y * (ga_v[sl] + gb_v[sl] + y * x_v[sl])
        h2 = jnp.zeros((LANE,), jnp.float32)
        for j in range(16):
            h2 = h2 + jnp.maximum(s1 * w1_v[j] + b1_v[j], 0.0) * w3_v[j]
        w_v[sl] = y * h2
        return carry

    lax.fori_loop(0, VECS, nbody, 0)
    pltpu.sync_copy(w_v, wsp.at[pl.ds(off, SLICE)])

    @pl.when(core == 0)
    def _():
        pltpu.sync_copy(w_v, w_hbm.at[pl.ds(off, SLICE)])

    plsc.subcore_barrier()

    def ebody(i, carry):
        c = wid + i * NW
        pltpu.sync_copy(src_hbm.at[pl.ds(c * CH * ROW, CH * ROW)], src_v)
        pltpu.sync_copy(dst_hbm.at[pl.ds(c * CH * ROW, CH * ROW)], dst_v)
        pltpu.sync_copy(wsp.at[src_v], msg_v)
        pltpu.sync_copy(msg_v, acc.at[dst_v], add=True)
        return carry

    lax.fori_loop(0, _n_my_chunks(wid), ebody, 0)
    plsc.subcore_barrier()
    pltpu.sync_copy(acc.at[pl.ds(off, SLICE)], w_v)
    pltpu.sync_copy(w_v, g_hbm.at[pl.ds(core * N_PAD + off, SLICE)])


def kernel(normalized_x, edge_index, W1, b1, W3, b3):
    x0 = normalized_x[:, 0]
    x0p = jnp.pad(x0, (0, N_PAD - N_NODES))
    src = edge_index[0]
    dst = edge_index[1]
    zeros = jnp.zeros((SLICE,), jnp.float32)
    ones = jnp.ones((CH * ROW,), jnp.float32)

    deg_p = _deg_kernel(dst, ones, zeros)
    g1_p, dis = _conv1_kernel(x0p, deg_p, src, dst, zeros)
    w1b = jnp.tile(W1.reshape(LANE, 1), (1, LANE))
    b1b = jnp.tile(b1.reshape(LANE, 1), (1, LANE))
    w3b = jnp.tile(W3.reshape(LANE, 1), (1, LANE))
    g2_p, wv = _conv2_kernel(x0p, dis, g1_p, src, dst, zeros, w1b, b1b, w3b)

    g2 = g2_p[:N_NODES] + g2_p[N_PAD:N_PAD + N_NODES]
    out = b3[0] + dis[:N_NODES] * (g2 + wv[:N_NODES])
    return out[:, None]
